# baseline (speedup 1.0000x reference)
import jax
import jax.numpy as jnp
from jax.experimental import pallas as pl
from jax.experimental.pallas import tpu as pltpu

B, N, F, D = 16, 128, 16, 64
K = 2


def _one_graph(adj_i, x, e_i, ws):
    (We1_0, We2_0, We3_0, be_0, Wn1_0, Wn2_0, bn_0,
     We1_1, We2_1, We3_1, be_1, Wn1_1, Wn2_1, bn_1,
     We1_2, We2_2, We3_2, be_2, Wn1_2, Wn2_2, bn_2,
     L1, b1, L2, b2, L3, b3) = ws
    adj = adj_i.astype(jnp.float32)
    ones_row = jnp.ones((1, N), jnp.float32)

    deg = jax.lax.dot_general(ones_row, adj, (((1,), (1,)), ((), ())))
    inv_deg = 1.0 / jnp.maximum(deg, 1.0)

    layers = ((We1_0, We2_0, We3_0, be_0, Wn1_0, Wn2_0, bn_0),
              (We1_1, We2_1, We3_1, be_1, Wn1_1, Wn2_1, bn_1),
              (We1_2, We2_2, We3_2, be_2, Wn1_2, Wn2_2, bn_2))

    e_f = None
    msum = None
    for l, (We1, We2, We3, be, Wn1, Wn2, bn) in enumerate(layers):
        if l == 0:
            e0 = e_i.reshape(N * N, F)
            e3 = jax.lax.dot_general(We3, e0, (((0,), (1,)), ((), ())))
        else:
            e3 = jax.lax.dot_general(We3, e_f, (((0,), (0,)), ((), ())))
        e3 = e3.reshape(F, N, N)
        src = jax.lax.dot_general(We1, x, (((0,), (1,)), ((), ()))) + be
        dst = jax.lax.dot_general(We2, x, (((0,), (1,)), ((), ())))
        t = e3 + src[:, :, None] + dst[:, None, :]
        e_new = jnp.maximum(t, 0.0) * adj[None, :, :]
        msum = jax.lax.dot_general(e_new.reshape(F * N, N), ones_row,
                                   (((1,), (1,)), ((), ()))
                                   ).reshape(F, N)
        if l < 2:
            ms = msum * inv_deg
            xn = jax.lax.dot_general(x, Wn1, (((1,), (0,)), ((), ())))
            xm = jax.lax.dot_general(ms, Wn2, (((0,), (0,)), ((), ())))
            x = jnp.maximum(xn + xm + bn, 0.0)
            e_f = e_new.reshape(F, N * N)

    h = jax.lax.dot_general(ones_row, msum, (((1,), (1,)), ((), ()))) \
        * (1.0 / (N * N))
    h1 = jnp.maximum(jnp.dot(h, L1) + b1, 0.0)
    h2 = jnp.maximum(jnp.dot(h1, L2) + b2, 0.0)
    return jnp.dot(h2, L3) + b3


def _fused_kernel(adj_ref, x_ref, ea_ref, eb_ref,
                  *w_and_out):
    w_refs, out_ref = w_and_out[:-1], w_and_out[-1]
    ws = tuple(w[...] for w in w_refs)
    e_refs = (ea_ref, eb_ref)
    outs = []
    for i in range(K):
        outs.append(_one_graph(adj_ref[i], x_ref[i], e_refs[i][0], ws)
                    .reshape(1, 1, 1))
    out_ref[...] = jnp.concatenate(outs, axis=0)


def kernel(edge_index, x, edge_attr,
           We1_0, We2_0, We3_0, be_0, Wn1_0, Wn2_0, bn_0,
           We1_1, We2_1, We3_1, be_1, Wn1_1, Wn2_1, bn_1,
           We1_2, We2_2, We3_2, be_2, Wn1_2, Wn2_2, bn_2,
           L1, b1, L2, b2, L3, b3):
    weights = [We1_0, We2_0, We3_0, be_0.reshape(F, 1), Wn1_0, Wn2_0,
               bn_0.reshape(1, D),
               We1_1, We2_1, We3_1, be_1.reshape(F, 1), Wn1_1, Wn2_1,
               bn_1.reshape(1, D),
               We1_2, We2_2, We3_2, be_2.reshape(F, 1), Wn1_2, Wn2_2,
               bn_2.reshape(1, D),
               L1, b1.reshape(1, F), L2, b2.reshape(1, F),
               L3.reshape(F, 1), b3.reshape(1, 1)]

    def const_spec(w):
        nd = w.ndim
        return pl.BlockSpec(w.shape, lambda s, _nd=nd: (0,) * _nd)

    def e_spec(i):
        return pl.BlockSpec((1, N, N, F),
                            lambda s, _i=i: (K * s + _i, 0, 0, 0))

    in_specs = [
        pl.BlockSpec((K, N, N), lambda s: (s, 0, 0)),
        pl.BlockSpec((K, N, D), lambda s: (s, 0, 0)),
    ] + [e_spec(i) for i in range(K)] + [const_spec(w) for w in weights]

    out = pl.pallas_call(
        _fused_kernel,
        grid=(B // K,),
        in_specs=in_specs,
        out_specs=pl.BlockSpec((K, 1, 1), lambda s: (s, 0, 0)),
        out_shape=jax.ShapeDtypeStruct((B, 1, 1), jnp.float32),
        compiler_params=pltpu.CompilerParams(
            dimension_semantics=("arbitrary",),
            vmem_limit_bytes=110 * 1024 * 1024,
        ),
    )(edge_index, x.astype(jnp.float32),
      edge_attr, edge_attr, *weights)
    return out.reshape(B)


# speedup vs baseline: 1.1600x; 1.1600x over previous
import jax
import jax.numpy as jnp
from jax.experimental import pallas as pl
from jax.experimental.pallas import tpu as pltpu

B, N, F, D = 16, 128, 16, 64


def _fused_kernel(adj_ref, x_ref, e_ref,
                  We1_0, We2_0, We3_0, be_0, Wn1_0, Wn2_0, bn_0,
                  We1_1, We2_1, We3_1, be_1, Wn1_1, Wn2_1, bn_1,
                  We1_2, We2_2, We3_2, be_2, Wn1_2, Wn2_2, bn_2,
                  L1, b1, L2, b2, L3, b3,
                  out_ref):
    adj = adj_ref[0].astype(jnp.float32)
    x = x_ref[0]
    ones_row = jnp.ones((1, N), jnp.float32)

    deg = jax.lax.dot_general(ones_row, adj, (((1,), (1,)), ((), ())))
    inv_deg = 1.0 / jnp.maximum(deg, 1.0)

    layers = ((We1_0, We2_0, We3_0, be_0, Wn1_0, Wn2_0, bn_0),
              (We1_1, We2_1, We3_1, be_1, Wn1_1, Wn2_1, bn_1),
              (We1_2, We2_2, We3_2, be_2, Wn1_2, Wn2_2, bn_2))

    e_f = None
    msum = None
    for l, (We1, We2, We3, be, Wn1, Wn2, bn) in enumerate(layers):
        if l == 0:
            e0 = e_ref[0].reshape(N * N, F).astype(jnp.float32)
            e3 = jax.lax.dot_general(We3[...], e0, (((0,), (1,)), ((), ())))
        else:
            e3 = jax.lax.dot_general(We3[...], e_f, (((0,), (0,)), ((), ())))
        e3 = e3.reshape(F, N, N)
        src = jax.lax.dot_general(We1[...], x, (((0,), (1,)), ((), ()))) \
            + be[...]
        dst = jax.lax.dot_general(We2[...], x, (((0,), (1,)), ((), ())))
        t = e3 + src[:, :, None] + dst[:, None, :]
        e_new = jnp.maximum(t, 0.0) * adj[None, :, :]
        msum = jax.lax.dot_general(e_new.reshape(F * N, N), ones_row,
                                   (((1,), (1,)), ((), ()))
                                   ).reshape(F, N)
        if l < 2:
            ms = msum * inv_deg
            xn = jax.lax.dot_general(x, Wn1[...], (((1,), (0,)), ((), ())))
            xm = jax.lax.dot_general(ms, Wn2[...], (((0,), (0,)), ((), ())))
            x = jnp.maximum(xn + xm + bn[...], 0.0)
            e_f = e_new.reshape(F, N * N)

    h = jax.lax.dot_general(ones_row, msum, (((1,), (1,)), ((), ()))) \
        * (1.0 / (N * N))
    h1 = jnp.maximum(jnp.dot(h, L1[...]) + b1[...], 0.0)
    h2 = jnp.maximum(jnp.dot(h1, L2[...]) + b2[...], 0.0)
    out = jnp.dot(h2, L3[...]) + b3[...]
    out_ref[...] = out.reshape(1, 1, 1)


def kernel(edge_index, x, edge_attr,
           We1_0, We2_0, We3_0, be_0, Wn1_0, Wn2_0, bn_0,
           We1_1, We2_1, We3_1, be_1, Wn1_1, Wn2_1, bn_1,
           We1_2, We2_2, We3_2, be_2, Wn1_2, Wn2_2, bn_2,
           L1, b1, L2, b2, L3, b3):
    weights = [We1_0, We2_0, We3_0, be_0.reshape(F, 1), Wn1_0, Wn2_0,
               bn_0.reshape(1, D),
               We1_1, We2_1, We3_1, be_1.reshape(F, 1), Wn1_1, Wn2_1,
               bn_1.reshape(1, D),
               We1_2, We2_2, We3_2, be_2.reshape(F, 1), Wn1_2, Wn2_2,
               bn_2.reshape(1, D),
               L1, b1.reshape(1, F), L2, b2.reshape(1, F),
               L3.reshape(F, 1), b3.reshape(1, 1)]

    def const_spec(w):
        nd = w.ndim
        return pl.BlockSpec(w.shape, lambda b, _nd=nd: (0,) * _nd)

    in_specs = [
        pl.BlockSpec((1, N, N), lambda b: (b, 0, 0)),
        pl.BlockSpec((1, N, D), lambda b: (b, 0, 0)),
        pl.BlockSpec((1, N, N, F), lambda b: (b, 0, 0, 0)),
    ] + [const_spec(w) for w in weights]

    out = pl.pallas_call(
        _fused_kernel,
        grid=(B,),
        in_specs=in_specs,
        out_specs=pl.BlockSpec((1, 1, 1), lambda b: (b, 0, 0)),
        out_shape=jax.ShapeDtypeStruct((B, 1, 1), jnp.float32),
        compiler_params=pltpu.CompilerParams(
            dimension_semantics=("arbitrary",),
        ),
    )(edge_index, x.astype(jnp.float32), edge_attr.astype(jnp.bfloat16),
      *weights)
    return out.reshape(B)
